# Initial kernel scaffold; baseline (speedup 1.0000x reference)
import jax
import jax.numpy as jnp
from jax.experimental import pallas as pl


def kernel(x, embeddings, W_e1, b_e1, W_e2, b_e2, W_q1, b_q1, W_q2, b_q2, W_d1, b_d1, W_d2, b_d2):
    raise NotImplementedError("write your pallas kernel here")


# speedup vs baseline: 2.9255x; 2.9255x over previous
import functools

import jax
import jax.numpy as jnp
from jax import lax
from jax.experimental import pallas as pl
from jax.experimental.pallas import tpu as pltpu
from jax.experimental.pallas import tpu_sc as plsc

SOM0, SOM1 = 32, 32
K = SOM0 * SOM1
LAT = 64
HID = 256
INP = 512
B = 1024
RB = 256
NBLK = B // RB
SENTINEL = K

_F32 = jnp.float32
_HIGH = lax.Precision.HIGHEST


def _dot(a, b, trans_b=False, precision=None):
    dn = (((1,), (1 if trans_b else 0,)), ((), ()))
    return lax.dot_general(a, b, dimension_numbers=dn,
                           precision=precision, preferred_element_type=_F32)


def _sigmoid(v):
    return 1.0 / (1.0 + jnp.exp(-v))


def _tc1_body(x_ref, e_ref, we1_ref, be1_ref, we2_ref, be2_ref,
              wd1_ref, bd1_ref, wd2_ref, bd2_ref,
              ze_ref, dist_ref, k_ref, idx8_ref, xe_ref):
    xb = x_ref[...]
    h = jnp.maximum(_dot(xb, we1_ref[...]) + be1_ref[...], 0.0)
    ze = jnp.maximum(_dot(h, we2_ref[...]) + be2_ref[...], 0.0)
    ze_ref[...] = ze

    e = e_ref[...]
    dots = _dot(ze, e, trans_b=True, precision=_HIGH)
    e2 = jnp.sum(e * e, axis=1)
    ze2 = jnp.sum(ze * ze, axis=1, keepdims=True)
    dist = ze2 - 2.0 * dots + e2[None, :]
    dist_ref[...] = dist

    m = jnp.min(dist, axis=1, keepdims=True)
    iota = lax.broadcasted_iota(jnp.int32, (RB, K), 1)
    idx = jnp.min(jnp.where(dist <= m, iota, jnp.int32(2 ** 30)), axis=1)
    k_ref[...] = idx.reshape(1, 1, RB)

    col = lax.broadcasted_iota(jnp.int32, (RB, 8), 1)
    kb = idx[:, None]
    k2 = lax.bitwise_and(kb, SOM1 - 1)
    sent = jnp.int32(SENTINEL)
    v = jnp.where(col == 1, jnp.where(kb < K - SOM1, kb + SOM1, sent), kb)
    v = jnp.where(col == 2, jnp.where(kb >= SOM1, kb - SOM1, sent), v)
    v = jnp.where(col == 3, jnp.where(k2 < SOM1 - 1, kb + 1, sent), v)
    v = jnp.where(col == 4, jnp.where(k2 > 0, kb - 1, sent), v)
    v = jnp.where(col >= 5, sent, v)
    idx8_ref[...] = v

    hd = jnp.maximum(_dot(ze, wd1_ref[...]) + bd1_ref[...], 0.0)
    xe_ref[...] = _sigmoid(_dot(hd, wd2_ref[...]) + bd2_ref[...])


def _tc1(x, e_flat, W_e1, b_e1, W_e2, b_e2, W_d1, b_d1, W_d2, b_d2):
    full = lambda shape: pl.BlockSpec(shape, lambda i: (0,) * len(shape))
    return pl.pallas_call(
        _tc1_body,
        grid=(NBLK,),
        in_specs=[
            pl.BlockSpec((RB, INP), lambda i: (i, 0)),
            full((K, LAT)),
            full((INP, HID)), full((1, HID)),
            full((HID, LAT)), full((1, LAT)),
            full((LAT, HID)), full((1, HID)),
            full((HID, INP)), full((1, INP)),
        ],
        out_specs=[
            pl.BlockSpec((RB, LAT), lambda i: (i, 0)),
            pl.BlockSpec((RB, K), lambda i: (i, 0)),
            pl.BlockSpec((1, 1, RB), lambda i: (i, 0, 0)),
            pl.BlockSpec((RB, 8), lambda i: (i, 0)),
            pl.BlockSpec((RB, INP), lambda i: (i, 0)),
        ],
        out_shape=[
            jax.ShapeDtypeStruct((B, LAT), _F32),
            jax.ShapeDtypeStruct((B, K), _F32),
            jax.ShapeDtypeStruct((NBLK, 1, RB), jnp.int32),
            jax.ShapeDtypeStruct((B, 8), jnp.int32),
            jax.ShapeDtypeStruct((B, INP), _F32),
        ],
    )(x, e_flat, W_e1, b_e1, W_e2, b_e2, W_d1, b_d1, W_d2, b_d2)


def _sc_gather(k_idx, idx5, table_pad):
    info = plsc.get_sparse_core_info()
    nc, ns = info.num_cores, info.num_subcores
    nw = nc * ns
    rows_w = B // nw
    nbr_h = rows_w * 5 // 2
    mesh = plsc.VectorSubcoreMesh(core_axis_name="c", subcore_axis_name="s")

    @functools.partial(
        pl.kernel,
        out_type=[
            jax.ShapeDtypeStruct((B, LAT), _F32),
            jax.ShapeDtypeStruct((B * 5, LAT), _F32),
        ],
        mesh=mesh,
        compiler_params=pltpu.CompilerParams(use_tc_tiling_on_sc=False),
        scratch_types=[
            pltpu.VMEM((rows_w,), jnp.int32),
            pltpu.VMEM((nbr_h,), jnp.int32),
            pltpu.VMEM((nbr_h,), jnp.int32),
            pltpu.VMEM((rows_w, LAT), _F32),
            pltpu.VMEM((nbr_h, LAT), _F32),
            pltpu.VMEM((nbr_h, LAT), _F32),
            pltpu.SemaphoreType.DMA,
            pltpu.SemaphoreType.DMA,
            pltpu.SemaphoreType.DMA,
        ],
    )
    def body(k_hbm, idx5_hbm, tab_hbm, zq_hbm, nbr_hbm,
             kv, nidx_a, nidx_b, crows, nrows_a, nrows_b, sem0, sem1, sem2):
        wid = lax.axis_index("s") * nc + lax.axis_index("c")
        base = wid * rows_w
        pltpu.sync_copy(k_hbm.at[pl.ds(base, rows_w)], kv)
        pltpu.sync_copy(idx5_hbm.at[pl.ds(base * 5, nbr_h)], nidx_a)
        pltpu.sync_copy(idx5_hbm.at[pl.ds(base * 5 + nbr_h, nbr_h)], nidx_b)
        cp0 = pltpu.async_copy(tab_hbm.at[kv], crows, sem0)
        cp1 = pltpu.async_copy(tab_hbm.at[nidx_a], nrows_a, sem1)
        cp2 = pltpu.async_copy(tab_hbm.at[nidx_b], nrows_b, sem2)
        cp0.wait()
        cp1.wait()
        cp2.wait()
        pltpu.sync_copy(crows, zq_hbm.at[pl.ds(base, rows_w)])
        pltpu.sync_copy(nrows_a, nbr_hbm.at[pl.ds(base * 5, nbr_h)])
        pltpu.sync_copy(nrows_b, nbr_hbm.at[pl.ds(base * 5 + nbr_h, nbr_h)])

    return body(k_idx, idx5, table_pad)


def _tc2_body(zq_ref, wq1_ref, bq1_ref, wq2_ref, bq2_ref, out_ref):
    hq = jnp.maximum(_dot(zq_ref[...], wq1_ref[...]) + bq1_ref[...], 0.0)
    out_ref[...] = _sigmoid(_dot(hq, wq2_ref[...]) + bq2_ref[...])


def _tc2(z_q, W_q1, b_q1, W_q2, b_q2):
    full = lambda shape: pl.BlockSpec(shape, lambda i: (0,) * len(shape))
    return pl.pallas_call(
        _tc2_body,
        grid=(NBLK,),
        in_specs=[
            pl.BlockSpec((RB, LAT), lambda i: (i, 0)),
            full((LAT, HID)), full((1, HID)),
            full((HID, INP)), full((1, INP)),
        ],
        out_specs=pl.BlockSpec((RB, INP), lambda i: (i, 0)),
        out_shape=jax.ShapeDtypeStruct((B, INP), _F32),
    )(z_q, W_q1, b_q1, W_q2, b_q2)


def kernel(x, embeddings, W_e1, b_e1, W_e2, b_e2, W_q1, b_q1, W_q2, b_q2,
           W_d1, b_d1, W_d2, b_d2):
    e_flat = embeddings.reshape(K, LAT)
    table_pad = jnp.concatenate([e_flat, jnp.zeros((8, LAT), _F32)], axis=0)

    z_e, z_dist_flat, k_blk, idx8, x_hat_e = _tc1(
        x, e_flat, W_e1, b_e1.reshape(1, HID), W_e2, b_e2.reshape(1, LAT),
        W_d1, b_d1.reshape(1, HID), W_d2, b_d2.reshape(1, INP))
    k = k_blk.reshape(B)
    idx5 = idx8[:, :5].reshape(B * 5)

    z_q, nbr = _sc_gather(k, idx5, table_pad)
    z_q_neighbors = nbr.reshape(B, 5, LAT)

    x_hat_q = _tc2(z_q, W_q1, b_q1.reshape(1, HID), W_q2, b_q2.reshape(1, INP))
    return (x_hat_q, x_hat_e, z_e, z_q, k, z_dist_flat, z_q_neighbors)
